# Initial kernel scaffold; baseline (speedup 1.0000x reference)
import jax
import jax.numpy as jnp
from jax.experimental import pallas as pl


def kernel(x, proposals, W1, b1, W2, b2, Wc, bc, Wb, bb):
    raise NotImplementedError("write your pallas kernel here")


# speedup vs baseline: 44.6526x; 44.6526x over previous
import functools

import jax
import jax.numpy as jnp
from jax.experimental import pallas as pl
from jax.experimental.pallas import tpu as pltpu

N = 1000
C = 91
CP = 128
IN_CH = 256 * 7 * 7
MID = 1024
K_BLK = 1792
K_STEPS = IN_CH // K_BLK
SCORE_THRESH = 0.1
NMS_THRESH = 0.6
MAX_DET = 100
IMG_SIZE = 1024.0
BBOX_XFORM_CLIP = float(jnp.log(1000.0 / 16.0))


def _head_body(x_ref, w1_ref, prop_ref, w2_ref, b1_ref, b2_ref, wc_ref,
               bc_ref, wb_ref, bb_ref,
               probs_ref, bx1_ref, by1_ref, bx2_ref, by2_ref,
               acc_ref):
    i = pl.program_id(0)

    @pl.when(i == 0)
    def _init():
        acc_ref[...] = jnp.zeros_like(acc_ref)

    acc_ref[...] += jnp.dot(x_ref[...], w1_ref[...],
                            preferred_element_type=jnp.float32)

    @pl.when(i == K_STEPS - 1)
    def _finish():
        h1 = jax.nn.relu(acc_ref[...] + b1_ref[...])
        h2 = jax.nn.relu(jnp.dot(h1, w2_ref[...],
                                 preferred_element_type=jnp.float32)
                         + b2_ref[...])
        logits = jnp.dot(h2, wc_ref[...],
                         preferred_element_type=jnp.float32) + bc_ref[...]
        m = jnp.max(logits, axis=-1, keepdims=True)
        e = jnp.exp(logits - m)
        probs_ref[...] = e / jnp.sum(e, axis=-1, keepdims=True)

        d = jnp.dot(h2, wb_ref[...],
                    preferred_element_type=jnp.float32) + bb_ref[...]
        dx = d[:, 0:CP] / 10.0
        dy = d[:, CP:2 * CP] / 10.0
        dw = jnp.minimum(d[:, 2 * CP:3 * CP] / 5.0, BBOX_XFORM_CLIP)
        dh = jnp.minimum(d[:, 3 * CP:4 * CP] / 5.0, BBOX_XFORM_CLIP)

        pw = prop_ref[:, 2:3] - prop_ref[:, 0:1]
        ph = prop_ref[:, 3:4] - prop_ref[:, 1:2]
        pcx = prop_ref[:, 0:1] + 0.5 * pw
        pcy = prop_ref[:, 1:2] + 0.5 * ph

        cx = dx * pw + pcx
        cy = dy * ph + pcy
        w = jnp.exp(dw) * pw
        hh = jnp.exp(dh) * ph
        bx1_ref[...] = jnp.clip(cx - 0.5 * w, 0.0, IMG_SIZE)
        by1_ref[...] = jnp.clip(cy - 0.5 * hh, 0.0, IMG_SIZE)
        bx2_ref[...] = jnp.clip(cx + 0.5 * w, 0.0, IMG_SIZE)
        by2_ref[...] = jnp.clip(cy + 0.5 * hh, 0.0, IMG_SIZE)


@functools.partial(jax.jit, static_argnames=())
def _head(x, proposals, W1, b1, W2, b2, Wc, bc, Wb, bb):
    wc_p = jnp.pad(Wc, ((0, 0), (0, CP - C)))
    bc_p = jnp.concatenate(
        [bc, jnp.full((CP - C,), -1e30, jnp.float32)]).reshape(1, CP)
    wb3 = Wb.reshape(MID, C, 4)
    wb_p = jnp.concatenate(
        [jnp.pad(wb3[:, :, k], ((0, 0), (0, CP - C))) for k in range(4)],
        axis=1)
    bb3 = bb.reshape(C, 4)
    bb_p = jnp.concatenate(
        [jnp.pad(bb3[:, k], (0, CP - C)) for k in range(4)]).reshape(1, 4 * CP)
    b1_p = b1.reshape(1, MID)
    b2_p = b2.reshape(1, MID)

    out_shapes = [jax.ShapeDtypeStruct((N, CP), jnp.float32)
                  for _ in range(5)]
    full = lambda shape: pl.BlockSpec(shape, lambda i: (0, 0))
    outs = pl.pallas_call(
        _head_body,
        grid=(K_STEPS,),
        in_specs=[
            pl.BlockSpec((N, K_BLK), lambda i: (0, i)),
            pl.BlockSpec((K_BLK, MID), lambda i: (i, 0)),
            full((N, 4)),
            full((MID, MID)),
            full((1, MID)), full((1, MID)),
            full((MID, CP)), full((1, CP)),
            full((MID, 4 * CP)), full((1, 4 * CP)),
        ],
        out_specs=[full((N, CP)) for _ in range(5)],
        out_shape=out_shapes,
        scratch_shapes=[pltpu.VMEM((N, MID), jnp.float32)],
    )(x, W1, proposals, W2, b1_p, b2_p, wc_p, bc_p, wb_p, bb_p)
    return outs


def _nms_all(b, s):
    area = (b[:, :, 2] - b[:, :, 0]) * (b[:, :, 3] - b[:, :, 1])
    idxs = jnp.arange(N)[None, :]

    def body(i, keep):
        bi = jax.lax.dynamic_slice_in_dim(b, i, 1, axis=1)
        ai = jax.lax.dynamic_slice_in_dim(area, i, 1, axis=1)
        lt = jnp.maximum(bi[:, :, :2], b[:, :, :2])
        rb = jnp.minimum(bi[:, :, 2:], b[:, :, 2:])
        wh = jnp.clip(rb - lt, 0.0, None)
        inter = wh[:, :, 0] * wh[:, :, 1]
        iou = inter / (ai + area - inter + 1e-9)
        sup = (iou > NMS_THRESH) & (idxs > i)
        ki = jax.lax.dynamic_slice_in_dim(keep, i, 1, axis=1)
        return keep & ~(sup & ki)

    keep = jax.lax.fori_loop(0, N, body, s > SCORE_THRESH)
    return jnp.where(keep, s, 0.0)


def kernel(x, proposals, W1, b1, W2, b2, Wc, bc, Wb, bb):
    probs, bx1, by1, bx2, by2 = _head(
        x, proposals, W1, b1, W2, b2, Wc, bc, Wb, bb)

    fg_scores = probs[:, 1:C].T
    fg_boxes = jnp.stack(
        [bx1[:, 1:C].T, by1[:, 1:C].T, bx2[:, 1:C].T, by2[:, 1:C].T],
        axis=-1)

    order = jnp.argsort(-fg_scores, axis=1)
    b = jnp.take_along_axis(fg_boxes, order[:, :, None], axis=1)
    s = jnp.take_along_axis(fg_scores, order, axis=1)

    s_out = _nms_all(b, s)

    labels = jnp.broadcast_to(jnp.arange(1, C)[:, None], s_out.shape)
    flat_s = s_out.reshape(-1)
    flat_b = b.reshape(-1, 4)
    flat_l = labels.reshape(-1)
    top_s, top_i = jax.lax.top_k(flat_s, MAX_DET)
    det_boxes = flat_b[top_i]
    det_labels = flat_l[top_i]
    return det_boxes, top_s, det_labels


# speedup vs baseline: 156.3268x; 3.5010x over previous
import functools

import jax
import jax.numpy as jnp
from jax.experimental import pallas as pl
from jax.experimental.pallas import tpu as pltpu

N = 1000
NP = 1024
C = 91
CP = 128
IN_CH = 256 * 7 * 7
MID = 1024
K_BLK = 1792
K_STEPS = IN_CH // K_BLK
SCORE_THRESH = 0.1
NMS_THRESH = 0.6
MAX_DET = 100
IMG_SIZE = 1024.0
BBOX_XFORM_CLIP = float(jnp.log(1000.0 / 16.0))


def _head_body(x_ref, w1_ref, prop_ref, w2_ref, b1_ref, b2_ref, wc_ref,
               bc_ref, wb_ref, bb_ref,
               probs_ref, bx1_ref, by1_ref, bx2_ref, by2_ref,
               acc_ref):
    i = pl.program_id(0)

    @pl.when(i == 0)
    def _init():
        acc_ref[...] = jnp.zeros_like(acc_ref)

    acc_ref[...] += jnp.dot(x_ref[...], w1_ref[...],
                            preferred_element_type=jnp.float32)

    @pl.when(i == K_STEPS - 1)
    def _finish():
        h1 = jax.nn.relu(acc_ref[...] + b1_ref[...])
        h2 = jax.nn.relu(jnp.dot(h1, w2_ref[...],
                                 preferred_element_type=jnp.float32)
                         + b2_ref[...])
        logits = jnp.dot(h2, wc_ref[...],
                         preferred_element_type=jnp.float32) + bc_ref[...]
        m = jnp.max(logits, axis=-1, keepdims=True)
        e = jnp.exp(logits - m)
        probs_ref[...] = e / jnp.sum(e, axis=-1, keepdims=True)

        d = jnp.dot(h2, wb_ref[...],
                    preferred_element_type=jnp.float32) + bb_ref[...]
        dx = d[:, 0:CP] / 10.0
        dy = d[:, CP:2 * CP] / 10.0
        dw = jnp.minimum(d[:, 2 * CP:3 * CP] / 5.0, BBOX_XFORM_CLIP)
        dh = jnp.minimum(d[:, 3 * CP:4 * CP] / 5.0, BBOX_XFORM_CLIP)

        pw = prop_ref[:, 2:3] - prop_ref[:, 0:1]
        ph = prop_ref[:, 3:4] - prop_ref[:, 1:2]
        pcx = prop_ref[:, 0:1] + 0.5 * pw
        pcy = prop_ref[:, 1:2] + 0.5 * ph

        cx = dx * pw + pcx
        cy = dy * ph + pcy
        w = jnp.exp(dw) * pw
        hh = jnp.exp(dh) * ph
        bx1_ref[...] = jnp.clip(cx - 0.5 * w, 0.0, IMG_SIZE)
        by1_ref[...] = jnp.clip(cy - 0.5 * hh, 0.0, IMG_SIZE)
        bx2_ref[...] = jnp.clip(cx + 0.5 * w, 0.0, IMG_SIZE)
        by2_ref[...] = jnp.clip(cy + 0.5 * hh, 0.0, IMG_SIZE)


def _head(x, proposals, W1, b1, W2, b2, Wc, bc, Wb, bb):
    wc_p = jnp.pad(Wc, ((0, 0), (0, CP - C)))
    bc_p = jnp.concatenate(
        [bc, jnp.full((CP - C,), -1e30, jnp.float32)]).reshape(1, CP)
    wb3 = Wb.reshape(MID, C, 4)
    wb_p = jnp.concatenate(
        [jnp.pad(wb3[:, :, k], ((0, 0), (0, CP - C))) for k in range(4)],
        axis=1)
    bb3 = bb.reshape(C, 4)
    bb_p = jnp.concatenate(
        [jnp.pad(bb3[:, k], (0, CP - C)) for k in range(4)]).reshape(1, 4 * CP)
    b1_p = b1.reshape(1, MID)
    b2_p = b2.reshape(1, MID)

    out_shapes = [jax.ShapeDtypeStruct((N, CP), jnp.float32)
                  for _ in range(5)]
    full = lambda shape: pl.BlockSpec(shape, lambda i: (0, 0))
    outs = pl.pallas_call(
        _head_body,
        grid=(K_STEPS,),
        in_specs=[
            pl.BlockSpec((N, K_BLK), lambda i: (0, i)),
            pl.BlockSpec((K_BLK, MID), lambda i: (i, 0)),
            full((N, 4)),
            full((MID, MID)),
            full((1, MID)), full((1, MID)),
            full((MID, CP)), full((1, CP)),
            full((MID, 4 * CP)), full((1, 4 * CP)),
        ],
        out_specs=[full((N, CP)) for _ in range(5)],
        out_shape=out_shapes,
        scratch_shapes=[pltpu.VMEM((N, MID), jnp.float32)],
    )(x, W1, proposals, W2, b1_p, b2_p, wc_p, bc_p, wb_p, bb_p)
    return outs


def _sort_nms_body(s_ref, x1_ref, y1_ref, x2_ref, y2_ref,
                   so_ref, x1o_ref, y1o_ref, x2o_ref, y2o_ref,
                   keep_ref, area_ref):
    s = s_ref[...]
    idx = jax.lax.broadcasted_iota(jnp.int32, (NP, CP), 0)
    payload = [x1_ref[...], y1_ref[...], x2_ref[...], y2_ref[...]]
    row = jax.lax.broadcasted_iota(jnp.int32, (NP, 1), 0)

    for k_ in [2 << t for t in range(10)]:
        for j in [k_ >> (t + 1) for t in range(k_.bit_length() - 1)]:
            mask_low = (row & j) == 0
            desc = (row & k_) == 0

            def partner(a):
                return jnp.where(mask_low, jnp.roll(a, -j, axis=0),
                                 jnp.roll(a, j, axis=0))

            sp = partner(s)
            ip = partner(idx)
            first = (s > sp) | ((s == sp) & (idx < ip))
            take = first ^ (mask_low == desc)
            s = jnp.where(take, sp, s)
            idx = jnp.where(take, ip, idx)
            payload = [jnp.where(take, partner(a), a) for a in payload]

    x1, y1, x2, y2 = payload
    x1o_ref[...] = x1
    y1o_ref[...] = y1
    x2o_ref[...] = x2
    y2o_ref[...] = y2
    area_ref[...] = (x2 - x1) * (y2 - y1)
    keep_ref[...] = (s > SCORE_THRESH).astype(jnp.float32)

    n_dyn = jnp.max(jnp.sum((s > SCORE_THRESH).astype(jnp.int32), axis=0))

    def body(i, carry):
        ki = keep_ref[pl.ds(i, 1), :]

        @pl.when(jnp.max(ki) > 0.0)
        def _do():
            xi1 = x1o_ref[pl.ds(i, 1), :]
            yi1 = y1o_ref[pl.ds(i, 1), :]
            xi2 = x2o_ref[pl.ds(i, 1), :]
            yi2 = y2o_ref[pl.ds(i, 1), :]
            ai = area_ref[pl.ds(i, 1), :]
            iw = jnp.clip(jnp.minimum(x2o_ref[...], xi2)
                          - jnp.maximum(x1o_ref[...], xi1), 0.0, None)
            ih = jnp.clip(jnp.minimum(y2o_ref[...], yi2)
                          - jnp.maximum(y1o_ref[...], yi1), 0.0, None)
            inter = iw * ih
            iou = inter / (ai + area_ref[...] - inter + 1e-9)
            sup = ((iou > NMS_THRESH) & (row > i)).astype(jnp.float32)
            keep_ref[...] = keep_ref[...] * (1.0 - sup * ki)

        return carry

    jax.lax.fori_loop(0, n_dyn, body, 0)
    so_ref[...] = jnp.where(keep_ref[...] > 0.0, s, 0.0)


def _sort_nms(scores, bx1, by1, bx2, by2):
    out_shapes = [jax.ShapeDtypeStruct((NP, CP), jnp.float32)
                  for _ in range(5)]
    return pl.pallas_call(
        _sort_nms_body,
        out_shape=out_shapes,
        scratch_shapes=[pltpu.VMEM((NP, CP), jnp.float32),
                        pltpu.VMEM((NP, CP), jnp.float32)],
    )(scores, bx1, by1, bx2, by2)


def kernel(x, proposals, W1, b1, W2, b2, Wc, bc, Wb, bb):
    probs, bx1, by1, bx2, by2 = _head(
        x, proposals, W1, b1, W2, b2, Wc, bc, Wb, bb)

    lane = jnp.arange(CP)[None, :]
    fg_lane = (lane >= 1) & (lane < C)
    s_in = jnp.where(fg_lane, probs, -1.0)
    pad_rows = lambda a, v: jnp.pad(a, ((0, NP - N), (0, 0)),
                                    constant_values=v)
    s_in = pad_rows(s_in, -1.0)
    so, x1s, y1s, x2s, y2s = _sort_nms(
        s_in, pad_rows(bx1, 0.0), pad_rows(by1, 0.0),
        pad_rows(bx2, 0.0), pad_rows(by2, 0.0))

    flat_s = so[:N, 1:C].T.reshape(-1)
    flat_b = jnp.stack([x1s[:N, 1:C].T, y1s[:N, 1:C].T,
                        x2s[:N, 1:C].T, y2s[:N, 1:C].T], axis=-1).reshape(-1, 4)
    flat_l = jnp.broadcast_to(jnp.arange(1, C)[:, None], (C - 1, N)).reshape(-1)
    top_s, top_i = jax.lax.top_k(flat_s, MAX_DET)
    det_boxes = flat_b[top_i]
    det_labels = flat_l[top_i]
    return det_boxes, top_s, det_labels


# speedup vs baseline: 225.5864x; 1.4430x over previous
import math

import jax
import jax.numpy as jnp
from jax.experimental import pallas as pl
from jax.experimental.pallas import tpu as pltpu

N = 1000
NP = 1024
C = 91
CP = 128
IN_CH = 256 * 7 * 7
MID = 1024
K_BLK = 1792
K_STEPS = IN_CH // K_BLK
SCORE_THRESH = 0.1
NMS_THRESH = 0.6
MAX_DET = 100
IMG_SIZE = 1024.0
BBOX_XFORM_CLIP = math.log(1000.0 / 16.0)


def _head_body(x_ref, w1_ref, prop_ref, w2_ref, b1_ref, b2_ref, wc_ref,
               bc_ref, wb_ref, bb_ref,
               probs_ref, bx1_ref, by1_ref, bx2_ref, by2_ref,
               acc_ref):
    i = pl.program_id(0)

    @pl.when(i == 0)
    def _init():
        acc_ref[...] = jnp.zeros_like(acc_ref)

    acc_ref[...] += jnp.dot(x_ref[...], w1_ref[...],
                            preferred_element_type=jnp.float32)

    @pl.when(i == K_STEPS - 1)
    def _finish():
        h1 = jax.nn.relu(acc_ref[...] + b1_ref[...])
        h2 = jax.nn.relu(jnp.dot(h1, w2_ref[...],
                                 preferred_element_type=jnp.float32)
                         + b2_ref[...])
        logits = jnp.dot(h2, wc_ref[...],
                         preferred_element_type=jnp.float32) + bc_ref[...]
        m = jnp.max(logits, axis=-1, keepdims=True)
        e = jnp.exp(logits - m)
        probs_ref[...] = e / jnp.sum(e, axis=-1, keepdims=True)

        d = jnp.dot(h2, wb_ref[...],
                    preferred_element_type=jnp.float32) + bb_ref[...]
        dx = d[:, 0:CP] / 10.0
        dy = d[:, CP:2 * CP] / 10.0
        dw = jnp.minimum(d[:, 2 * CP:3 * CP] / 5.0, BBOX_XFORM_CLIP)
        dh = jnp.minimum(d[:, 3 * CP:4 * CP] / 5.0, BBOX_XFORM_CLIP)

        pw = prop_ref[:, 2:3] - prop_ref[:, 0:1]
        ph = prop_ref[:, 3:4] - prop_ref[:, 1:2]
        pcx = prop_ref[:, 0:1] + 0.5 * pw
        pcy = prop_ref[:, 1:2] + 0.5 * ph

        cx = dx * pw + pcx
        cy = dy * ph + pcy
        w = jnp.exp(dw) * pw
        hh = jnp.exp(dh) * ph
        bx1_ref[...] = jnp.clip(cx - 0.5 * w, 0.0, IMG_SIZE)
        by1_ref[...] = jnp.clip(cy - 0.5 * hh, 0.0, IMG_SIZE)
        bx2_ref[...] = jnp.clip(cx + 0.5 * w, 0.0, IMG_SIZE)
        by2_ref[...] = jnp.clip(cy + 0.5 * hh, 0.0, IMG_SIZE)


def _head(x, proposals, W1, b1, W2, b2, Wc, bc, Wb, bb):
    wc_p = jnp.pad(Wc, ((0, 0), (0, CP - C)))
    bc_p = jnp.concatenate(
        [bc, jnp.full((CP - C,), -1e30, jnp.float32)]).reshape(1, CP)
    wb3 = Wb.reshape(MID, C, 4)
    wb_p = jnp.concatenate(
        [jnp.pad(wb3[:, :, k], ((0, 0), (0, CP - C))) for k in range(4)],
        axis=1)
    bb3 = bb.reshape(C, 4)
    bb_p = jnp.concatenate(
        [jnp.pad(bb3[:, k], (0, CP - C)) for k in range(4)]).reshape(1, 4 * CP)
    b1_p = b1.reshape(1, MID)
    b2_p = b2.reshape(1, MID)

    out_shapes = [jax.ShapeDtypeStruct((N, CP), jnp.float32)
                  for _ in range(5)]
    full = lambda shape: pl.BlockSpec(shape, lambda i: (0, 0))
    outs = pl.pallas_call(
        _head_body,
        grid=(K_STEPS,),
        in_specs=[
            pl.BlockSpec((N, K_BLK), lambda i: (0, i)),
            pl.BlockSpec((K_BLK, MID), lambda i: (i, 0)),
            full((N, 4)),
            full((MID, MID)),
            full((1, MID)), full((1, MID)),
            full((MID, CP)), full((1, CP)),
            full((MID, 4 * CP)), full((1, 4 * CP)),
        ],
        out_specs=[full((N, CP)) for _ in range(5)],
        out_shape=out_shapes,
        scratch_shapes=[pltpu.VMEM((N, MID), jnp.float32)],
    )(x, W1, proposals, W2, b1_p, b2_p, wc_p, bc_p, wb_p, bb_p)
    return outs


def _stage(arrs, first_fn, roll, axis, mask_low, desc):
    def partner(a):
        return jnp.where(mask_low, jnp.roll(a, -roll, axis=axis),
                         jnp.roll(a, roll, axis=axis))

    parts = [partner(a) for a in arrs]
    first = first_fn(arrs, parts)
    take = first ^ (mask_low == desc)
    return [jnp.where(take, p, a) for a, p in zip(arrs, parts)]


def _sort_nms_body(s_ref, x1_ref, y1_ref, x2_ref, y2_ref,
                   topv_ref, topk_ref, tx1_ref, ty1_ref, tx2_ref, ty2_ref,
                   keep_ref, area_ref, x1o_ref, y1o_ref, x2o_ref, y2o_ref):
    s = s_ref[...]
    idx = jax.lax.broadcasted_iota(jnp.int32, (NP, CP), 0)
    row = jax.lax.broadcasted_iota(jnp.int32, (NP, 1), 0)

    def first_desc_score(arrs_, parts_):
        (s_, i_), (sp_, ip_) = arrs_[:2], parts_[:2]
        return (s_ > sp_) | ((s_ == sp_) & (i_ < ip_))

    arrs = [s, idx, x1_ref[...], y1_ref[...], x2_ref[...], y2_ref[...]]
    for k_ in [2 << t for t in range(10)]:
        for j in [k_ >> (t + 1) for t in range(k_.bit_length() - 1)]:
            arrs = _stage(arrs, first_desc_score, j, 0,
                          (row & j) == 0, (row & k_) == 0)
    s, x1, y1, x2, y2 = arrs[0], *arrs[2:]

    x1o_ref[...] = x1
    y1o_ref[...] = y1
    x2o_ref[...] = x2
    y2o_ref[...] = y2
    area_ref[...] = (x2 - x1) * (y2 - y1)
    keep_ref[...] = (s > SCORE_THRESH).astype(jnp.float32)

    n_dyn = jnp.max(jnp.sum((s > SCORE_THRESH).astype(jnp.int32), axis=0))

    def body(i, carry):
        ki = keep_ref[pl.ds(i, 1), :]

        @pl.when(jnp.max(ki) > 0.0)
        def _do():
            xi1 = x1o_ref[pl.ds(i, 1), :]
            yi1 = y1o_ref[pl.ds(i, 1), :]
            xi2 = x2o_ref[pl.ds(i, 1), :]
            yi2 = y2o_ref[pl.ds(i, 1), :]
            ai = area_ref[pl.ds(i, 1), :]
            iw = jnp.clip(jnp.minimum(x2o_ref[...], xi2)
                          - jnp.maximum(x1o_ref[...], xi1), 0.0, None)
            ih = jnp.clip(jnp.minimum(y2o_ref[...], yi2)
                          - jnp.maximum(y1o_ref[...], yi1), 0.0, None)
            inter = iw * ih
            iou = inter / (ai + area_ref[...] - inter + 1e-9)
            sup = ((iou > NMS_THRESH) & (row > i)).astype(jnp.float32)
            keep_ref[...] = keep_ref[...] * (1.0 - sup * ki)

        return carry

    jax.lax.fori_loop(0, n_dyn, body, 0)
    so = jnp.where(keep_ref[...] > 0.0, s, 0.0)

    pos0 = jax.lax.broadcasted_iota(jnp.int32, (NP, CP), 0)

    def first_compress(arrs_, parts_):
        (s_, p_), (sp_, pp_) = arrs_[:2], parts_[:2]
        zs = s_ == 0.0
        zp = sp_ == 0.0
        return (~zs & zp) | ((zs == zp) & (p_ < pp_))

    arrs = [so, pos0, x1, y1, x2, y2]
    for k_ in [2 << t for t in range(10)]:
        for j in [k_ >> (t + 1) for t in range(k_.bit_length() - 1)]:
            arrs = _stage(arrs, first_compress, j, 0,
                          (row & j) == 0, (row & k_) == 0)

    B = 128
    bpos = arrs[1][:B, :]
    brow = jax.lax.broadcasted_iota(jnp.int32, (B, 1), 0)
    blane = jax.lax.broadcasted_iota(jnp.int32, (1, CP), 1)
    valid = (blane >= 1) & (blane < C)
    fk = jnp.where(valid, (blane - 1) * N + bpos,
                   jnp.full_like(bpos, 100000000))

    def first_topk(arrs_, parts_):
        (s_, f_), (sp_, fp_) = arrs_[:2], parts_[:2]
        return (s_ > sp_) | ((s_ == sp_) & (f_ < fp_))

    arrs = [arrs[0][:B, :], fk] + [a[:B, :] for a in arrs[2:]]
    for k_ in [2 << t for t in range(14)]:
        kdesc = ((brow & (k_ // B)) == 0 if k_ >= B
                 else (blane & k_) == 0)
        for j in [k_ >> (t + 1) for t in range(k_.bit_length() - 1)]:
            if j >= B:
                arrs = _stage(arrs, first_topk, j // B, 0,
                              (brow & (j // B)) == 0, kdesc)
            else:
                arrs = _stage(arrs, first_topk, j, 1,
                              (blane & j) == 0, kdesc)

    topv_ref[...] = arrs[0][0:1, :]
    topk_ref[...] = arrs[1][0:1, :]
    tx1_ref[...] = arrs[2][0:1, :]
    ty1_ref[...] = arrs[3][0:1, :]
    tx2_ref[...] = arrs[4][0:1, :]
    ty2_ref[...] = arrs[5][0:1, :]


def _sort_nms(scores, bx1, by1, bx2, by2):
    out_shapes = [jax.ShapeDtypeStruct((1, CP), jnp.float32),
                  jax.ShapeDtypeStruct((1, CP), jnp.int32)] + \
                 [jax.ShapeDtypeStruct((1, CP), jnp.float32)
                  for _ in range(4)]
    return pl.pallas_call(
        _sort_nms_body,
        out_shape=out_shapes,
        scratch_shapes=[pltpu.VMEM((NP, CP), jnp.float32)
                        for _ in range(6)],
    )(scores, bx1, by1, bx2, by2)


def kernel(x, proposals, W1, b1, W2, b2, Wc, bc, Wb, bb):
    probs, bx1, by1, bx2, by2 = _head(
        x, proposals, W1, b1, W2, b2, Wc, bc, Wb, bb)

    lane = jnp.arange(CP)[None, :]
    fg_lane = (lane >= 1) & (lane < C)
    s_in = jnp.where(fg_lane, probs, -1.0)
    pad_rows = lambda a, v: jnp.pad(a, ((0, NP - N), (0, 0)),
                                    constant_values=v)
    topv, topk, tx1, ty1, tx2, ty2 = _sort_nms(
        pad_rows(s_in, -1.0), pad_rows(bx1, 0.0), pad_rows(by1, 0.0),
        pad_rows(bx2, 0.0), pad_rows(by2, 0.0))

    top_s = topv[0, :MAX_DET]
    det_boxes = jnp.stack([tx1[0, :MAX_DET], ty1[0, :MAX_DET],
                           tx2[0, :MAX_DET], ty2[0, :MAX_DET]], axis=-1)
    det_labels = (topk[0, :MAX_DET] // N + 1).astype(jnp.int32)
    return det_boxes, top_s, det_labels
